# baseline (speedup 1.0000x reference)
import functools

import jax
import jax.numpy as jnp
from jax import lax
from jax.experimental import pallas as pl
from jax.experimental.pallas import tpu as pltpu
from jax.experimental.pallas import tpu_sc as plsc


def kernel(indices, time_encodings):
    B, = indices.shape
    V, D = time_encodings.shape
    DP = 128
    L = 16

    info = plsc.get_sparse_core_info()
    NC, NS = info.num_cores, info.num_subcores
    NW = NC * NS
    b_per_w = B // NW
    NCH = 4
    CW = b_per_w // NCH

    mesh = plsc.VectorSubcoreMesh(core_axis_name="c", subcore_axis_name="s")

    @functools.partial(
        pl.kernel,
        mesh=mesh,
        out_type=jax.ShapeDtypeStruct((D, B), jnp.float32),
        scratch_types=[
            pltpu.VMEM((b_per_w,), jnp.int32),
            pltpu.VMEM((b_per_w, DP), jnp.float32),
            pltpu.VMEM((NCH * D, CW), jnp.float32),
            pltpu.SemaphoreType.DMA,
            pltpu.SemaphoreType.DMA,
        ],
        compiler_params=pltpu.CompilerParams(needs_layout_passes=False),
    )
    def gather_kernel(table_hbm, idx_hbm, out_hbm, idx_v, rows_v, t_v,
                      gsem, wsem):
        wid = lax.axis_index("s") * NC + lax.axis_index("c")
        base = wid * b_per_w
        pltpu.sync_copy(idx_hbm.at[pl.ds(base, b_per_w)], idx_v)
        gathers = [
            pltpu.async_copy(
                table_hbm.at[idx_v.at[pl.ds(c * CW, CW)]],
                rows_v.at[pl.ds(c * CW, CW)], gsem)
            for c in range(NCH)
        ]
        iota = lax.iota(jnp.int32, L)
        zeros = jnp.zeros((L,), jnp.int32)
        writes = []
        for c in range(NCH):
            gathers[c].wait()

            @plsc.parallel_loop(0, (D // L) * (CW // L) * 4, unroll=4)
            def transpose_block(i, c=c):
                g = i // ((CW // L) * 4)
                r = i % ((CW // L) * 4)
                b0 = (r // 4) * L
                s0 = (r % 4) * 4
                d_vec = iota + g * L
                for s in range(4):
                    b_vec = ((iota + s0 + s) & (L - 1)) + b0
                    vals = plsc.load_gather(
                        rows_v, [b_vec + c * CW, d_vec])
                    plsc.store_scatter(
                        t_v, [d_vec + c * D, b_vec], vals)
            writes.append(pltpu.async_copy(
                t_v.at[pl.ds(c * D, D)],
                out_hbm.at[:, pl.ds(base + c * CW, CW)], wsem))
        for w in writes:
            w.wait()

    table_pad = jnp.concatenate(
        [time_encodings, jnp.zeros((V, DP - D), jnp.float32)], axis=1)
    out_t = gather_kernel(table_pad, indices.astype(jnp.int32))
    return out_t.T


# speedup vs baseline: 1.0315x; 1.0315x over previous
import functools

import jax
import jax.numpy as jnp
from jax import lax
from jax.experimental import pallas as pl
from jax.experimental.pallas import tpu as pltpu
from jax.experimental.pallas import tpu_sc as plsc


def kernel(indices, time_encodings):
    B, = indices.shape
    V, D = time_encodings.shape
    DP = 128
    L = 16

    info = plsc.get_sparse_core_info()
    NC, NS = info.num_cores, info.num_subcores
    NW = NC * NS
    b_per_w = B // NW
    NCH = 4
    CW = b_per_w // NCH

    mesh = plsc.VectorSubcoreMesh(core_axis_name="c", subcore_axis_name="s")

    @functools.partial(
        pl.kernel,
        mesh=mesh,
        out_type=jax.ShapeDtypeStruct((D, B), jnp.float32),
        scratch_types=[
            pltpu.VMEM((b_per_w,), jnp.int32),
            pltpu.VMEM((b_per_w, DP), jnp.float32),
            pltpu.VMEM((NCH * D, CW), jnp.float32),
            pltpu.SemaphoreType.DMA,
            pltpu.SemaphoreType.DMA,
        ],
        compiler_params=pltpu.CompilerParams(needs_layout_passes=False),
    )
    def gather_kernel(table_hbm, idx_hbm, out_hbm, idx_v, rows_v, t_v,
                      gsem, wsem):
        wid = lax.axis_index("s") * NC + lax.axis_index("c")
        base = wid * b_per_w
        pltpu.sync_copy(idx_hbm.at[pl.ds(base, b_per_w)], idx_v)
        gathers = [
            pltpu.async_copy(
                table_hbm.at[idx_v.at[pl.ds(c * CW, CW)]],
                rows_v.at[pl.ds(c * CW, CW)], gsem)
            for c in range(NCH)
        ]
        iota = lax.iota(jnp.int32, L)
        zeros = jnp.zeros((L,), jnp.int32)
        writes = []
        for c in range(NCH):
            gathers[c].wait()

            @plsc.parallel_loop(0, (D // L) * (CW // L) * 4, unroll=2)
            def transpose_block(i, c=c):
                g = i // ((CW // L) * 4)
                r = i % ((CW // L) * 4)
                b0 = (r // 4) * L
                s0 = (r % 4) * 4
                d_vec = iota + g * L
                b_vecs = [((iota + s0 + s) & (L - 1)) + b0 for s in range(4)]
                vals = [plsc.load_gather(rows_v, [bv + c * CW, d_vec])
                        for bv in b_vecs]
                for s in range(4):
                    plsc.store_scatter(
                        t_v, [d_vec + c * D, b_vecs[s]], vals[s])
            writes.append(pltpu.async_copy(
                t_v.at[pl.ds(c * D, D)],
                out_hbm.at[:, pl.ds(base + c * CW, CW)], wsem))
        for w in writes:
            w.wait()

    table_pad = jnp.concatenate(
        [time_encodings, jnp.zeros((V, DP - D), jnp.float32)], axis=1)
    out_t = gather_kernel(table_pad, indices.astype(jnp.int32))
    return out_t.T
